# Initial kernel scaffold; baseline (speedup 1.0000x reference)
import jax
import jax.numpy as jnp
from jax.experimental import pallas as pl


def kernel(pcds_4d_batch, W0, b0, W1, b1, W2, b2, W3, b3):
    raise NotImplementedError("write your pallas kernel here")


# speedup vs baseline: 2.1829x; 2.1829x over previous
import functools

import jax
import jax.numpy as jnp
import numpy as np
from jax import lax
from jax.experimental import pallas as pl
from jax.experimental.pallas import tpu as pltpu
from jax.experimental.pallas import tpu_sc as plsc

M_BUCKETS = 1 << 20
MASK = M_BUCKETS - 1
HP = (73856093, 19349669, 83492791, 49979693, 15485863)
DELTAS = (73856093, -73856093, 19349669, -19349669, 83492791, -83492791)

NC, NS, L = 2, 16, 16
NW = NC * NS
C = 2048
ZCH = M_BUCKETS // NS


def _floor_div(x, q):
    d = x / q
    t = d.astype(jnp.int32)
    tf = t.astype(jnp.float32)
    return jnp.where(tf > d, t - 1, t)


def _hash_chunk(pbuf, hbuf, base, i, n_batch, n_total):
    o = i * L
    xs = pbuf[pl.ds(0 * C + o, L)]
    ys = pbuf[pl.ds(1 * C + o, L)]
    zs = pbuf[pl.ds(2 * C + o, L)]
    ts = pbuf[pl.ds(3 * C + o, L)]
    c0 = _floor_div(xs, 0.1)
    c1 = _floor_div(ys, 0.1)
    c2 = _floor_div(zs, 0.1)
    c3 = _floor_div(ts, 1.0)
    gidx = base + o + lax.iota(jnp.int32, L)
    hb = jnp.zeros((L,), jnp.int32)
    b = n_batch
    while b < n_total:
        hb = hb + jnp.where(gidx >= b, HP[4], 0)
        b += n_batch
    h = (c0 * HP[0] + c1 * HP[1] + c2 * HP[2] + c3 * HP[3] + hb)
    hbuf[pl.ds(o, L)] = h & MASK


def _make_sc_kernel(n_total, n_batch, m_out):
    own = ((n_total + NW - 1) // NW + L - 1) // L * L
    srange = ((n_total + NS - 1) // NS + L - 1) // L * L
    n_own_ch = (own + C - 1) // C
    n_sc_ch = (srange + C - 1) // C
    mesh = plsc.VectorSubcoreMesh(core_axis_name="c", subcore_axis_name="s")

    def body(pts_hbm, idx_hbm, pbuf, hbuf, ones, zbuf, ilist, taps, patt, counts):
        cid = lax.axis_index("c")
        sid = lax.axis_index("s")
        wid = sid * NC + cid


        def fill(i, _):
            zbuf[pl.ds(i * L, L)] = jnp.zeros((L,), jnp.float32)
            ones[pl.ds(i * L, L)] = jnp.full((L,), 1.0, jnp.float32)
            return 0

        lax.fori_loop(0, C // L, fill, 0)

        def zero_counts(j, _):
            pltpu.sync_copy(zbuf, counts.at[pl.ds(sid * ZCH + j * C, C)])
            return 0

        lax.fori_loop(0, ZCH // C, zero_counts, 0)
        plsc.subcore_barrier()


        sstart = jnp.minimum(sid * srange, n_total - srange)

        def scatter_chunk(j, _):
            base = sstart + jnp.minimum(j * C, srange - C)
            for r in range(4):
                pltpu.sync_copy(pts_hbm.at[pl.ds(r * n_total + base, C)],
                                pbuf.at[pl.ds(r * C, C)])

            def hash_i(i, _):
                _hash_chunk(pbuf, hbuf, base, i, n_batch, n_total)
                return 0

            lax.fori_loop(0, C // L, hash_i, 0)
            pltpu.sync_copy(ones, counts.at[hbuf], add=True)
            return 0

        lax.fori_loop(0, n_sc_ch, scatter_chunk, 0)
        plsc.subcore_barrier()


        ostart = jnp.minimum(wid * own, n_total - own)

        def gather_chunk(j, _):
            base = ostart + jnp.minimum(j * C, own - C)
            for r in range(4):
                pltpu.sync_copy(pts_hbm.at[pl.ds(r * n_total + base, C)],
                                pbuf.at[pl.ds(r * C, C)])

            def hash_i(i, _):
                _hash_chunk(pbuf, hbuf, base, i, n_batch, n_total)
                o = i * L
                hv = hbuf[pl.ds(o, L)]
                for k in range(6):
                    ilist[pl.ds(k * C + o, L)] = (hv + DELTAS[k]) & MASK
                return 0

            lax.fori_loop(0, C // L, hash_i, 0)
            pltpu.sync_copy(counts.at[ilist], taps)

            def patt_i(i, _):
                o = i * L
                pv = jnp.full((L,), 1, jnp.int32)
                for k in range(6):
                    tv = taps[pl.ds(k * C + o, L)]
                    pv = pv | jnp.where(tv > 0.0, 1 << (k + 1), 0)
                patt[pl.ds(o, L)] = pv
                return 0

            lax.fori_loop(0, C // L, patt_i, 0)
            pltpu.sync_copy(patt, idx_hbm.at[pl.ds(base, C)])
            return 0

        lax.fori_loop(0, n_own_ch, gather_chunk, 0)

    return pl.kernel(
        body,
        out_type=jax.ShapeDtypeStruct((m_out,), jnp.int32),
        mesh=mesh,
        scratch_types=[
            pltpu.VMEM((4 * C,), jnp.float32),
            pltpu.VMEM((C,), jnp.int32),
            pltpu.VMEM((C,), jnp.float32),
            pltpu.VMEM((C,), jnp.float32),
            pltpu.VMEM((6 * C,), jnp.int32),
            pltpu.VMEM((6 * C,), jnp.float32),
            pltpu.VMEM((C,), jnp.int32),
            pltpu.VMEM_SHARED((M_BUCKETS,), jnp.float32),
        ],
    )


def _feats_body(idx_ref, w0, b0, w1, b1, w2, b2, w3, b3, out_ref, lut_ref):
    @pl.when(pl.program_id(0) == 0)
    def _():
        p = lax.broadcasted_iota(jnp.int32, (128, 7), 0)
        k = lax.broadcasted_iota(jnp.int32, (128, 7), 1)
        pat = (((p >> k) & 1).astype(jnp.float32)) * 0.5
        x = jnp.maximum(jnp.dot(pat, w0[...], preferred_element_type=jnp.float32)
                        + b0[...], 0.0)
        x = jnp.maximum(jnp.dot(x, w1[...], preferred_element_type=jnp.float32)
                        + b1[...], 0.0)
        x = jnp.maximum(jnp.dot(x, w2[...], preferred_element_type=jnp.float32)
                        + b2[...], 0.0)
        lut_ref[...] = (jnp.dot(x, w3[...], preferred_element_type=jnp.float32)
                        + b3[...])

    idxv = idx_ref[0, 0, :].reshape(-1, 1)
    oh = (idxv == lax.broadcasted_iota(jnp.int32, (idxv.shape[0], 128), 1)
          ).astype(jnp.float32)
    out_ref[...] = jnp.dot(oh, lut_ref[...], preferred_element_type=jnp.float32)


def _cs_body(x_ref, o_ref):
    m = lax.broadcasted_iota(jnp.int32, x_ref.shape, 1) % 4
    q = jnp.where(m == 3, 1.0, 0.1).astype(jnp.float32)
    o_ref[...] = jnp.floor(x_ref[...] / q) * q


def kernel(pcds_4d_batch, W0, b0, W1, b1, W2, b2, W3, b3):
    B, N, _ = pcds_4d_batch.shape
    n_total = B * N
    flat = pcds_4d_batch.reshape(n_total, 4)
    pts_t = flat.T.reshape(-1)

    BLK = 4096
    nb = (n_total + BLK - 1) // BLK
    m_out = nb * BLK

    patt_idx = _make_sc_kernel(n_total, N, m_out)(pts_t)

    feats = pl.pallas_call(
        _feats_body,
        grid=(nb,),
        in_specs=[
            pl.BlockSpec((1, 1, BLK), lambda i: (i, 0, 0)),
            pl.BlockSpec((7, 32), lambda i: (0, 0)),
            pl.BlockSpec((1, 32), lambda i: (0, 0)),
            pl.BlockSpec((32, 128), lambda i: (0, 0)),
            pl.BlockSpec((1, 128), lambda i: (0, 0)),
            pl.BlockSpec((128, 32), lambda i: (0, 0)),
            pl.BlockSpec((1, 32), lambda i: (0, 0)),
            pl.BlockSpec((32, 16), lambda i: (0, 0)),
            pl.BlockSpec((1, 16), lambda i: (0, 0)),
        ],
        out_specs=pl.BlockSpec((BLK, 16), lambda i: (i, 0)),
        out_shape=jax.ShapeDtypeStruct((m_out, 16), jnp.float32),
        scratch_shapes=[pltpu.VMEM((128, 16), jnp.float32)],
    )(patt_idx.reshape(nb, 1, BLK), W0, b0.reshape(1, 32), W1, b1.reshape(1, 128),
      W2, b2.reshape(1, 32), W3, b3.reshape(1, 16))
    point_feats = feats[:n_total]

    rows = n_total * 4 // 128
    RB = 4096
    nrb = (rows + RB - 1) // RB
    cs = pl.pallas_call(
        _cs_body,
        grid=(nrb,),
        in_specs=[pl.BlockSpec((RB, 128), lambda i: (i, 0))],
        out_specs=pl.BlockSpec((RB, 128), lambda i: (i, 0)),
        out_shape=jax.ShapeDtypeStruct((rows, 128), jnp.float32),
    )(flat.reshape(rows, 128))
    coords_scaled = cs.reshape(n_total, 4)

    return point_feats, coords_scaled


# speedup vs baseline: 2.3518x; 1.0774x over previous
import functools

import jax
import jax.numpy as jnp
import numpy as np
from jax import lax
from jax.experimental import pallas as pl
from jax.experimental.pallas import tpu as pltpu
from jax.experimental.pallas import tpu_sc as plsc

M_BUCKETS = 1 << 20
MASK = M_BUCKETS - 1
HP = (73856093, 19349669, 83492791, 49979693, 15485863)
DELTAS = (73856093, -73856093, 19349669, -19349669, 83492791, -83492791)

NC, NS, L = 2, 16, 16
NW = NC * NS
C = 2048
ZCH = M_BUCKETS // NS


def _floor_div(x, q):
    d = x / q
    t = d.astype(jnp.int32)
    tf = t.astype(jnp.float32)
    return jnp.where(tf > d, t - 1, t)


def _hash_chunk(pbuf, hbuf, base, i, n_batch, n_total):
    o = i * L
    xs = pbuf[pl.ds(0 * C + o, L)]
    ys = pbuf[pl.ds(1 * C + o, L)]
    zs = pbuf[pl.ds(2 * C + o, L)]
    ts = pbuf[pl.ds(3 * C + o, L)]
    c0 = _floor_div(xs, 0.1)
    c1 = _floor_div(ys, 0.1)
    c2 = _floor_div(zs, 0.1)
    c3 = _floor_div(ts, 1.0)
    gidx = base + o + lax.iota(jnp.int32, L)
    hb = jnp.zeros((L,), jnp.int32)
    b = n_batch
    while b < n_total:
        hb = hb + jnp.where(gidx >= b, HP[4], 0)
        b += n_batch
    h = (c0 * HP[0] + c1 * HP[1] + c2 * HP[2] + c3 * HP[3] + hb)
    hbuf[pl.ds(o, L)] = h & MASK


def _make_sc_kernel(n_total, n_batch, m_out):
    own = ((n_total + NW - 1) // NW + L - 1) // L * L
    srange = ((n_total + NS - 1) // NS + L - 1) // L * L
    n_own_ch = (own + C - 1) // C
    n_sc_ch = (srange + C - 1) // C
    mesh = plsc.VectorSubcoreMesh(core_axis_name="c", subcore_axis_name="s")

    def body(pts_hbm, idx_hbm, pbuf, hbuf, ones, zbuf, ilist, taps, patt, counts):
        cid = lax.axis_index("c")
        sid = lax.axis_index("s")
        wid = sid * NC + cid


        def fill(i, _):
            zbuf[pl.ds(i * L, L)] = jnp.zeros((L,), jnp.float32)
            ones[pl.ds(i * L, L)] = jnp.full((L,), 1.0, jnp.float32)
            return 0

        lax.fori_loop(0, C // L, fill, 0)

        def zero_counts(j, _):
            pltpu.sync_copy(zbuf, counts.at[pl.ds(sid * ZCH + j * C, C)])
            return 0

        lax.fori_loop(0, ZCH // C, zero_counts, 0)
        plsc.subcore_barrier()


        sstart = jnp.minimum(sid * srange, n_total - srange)

        def scatter_chunk(j, _):
            base = sstart + jnp.minimum(j * C, srange - C)
            for r in range(4):
                pltpu.sync_copy(pts_hbm.at[pl.ds(r * n_total + base, C)],
                                pbuf.at[pl.ds(r * C, C)])

            def hash_i(i, _):
                _hash_chunk(pbuf, hbuf, base, i, n_batch, n_total)
                return 0

            lax.fori_loop(0, C // L, hash_i, 0)
            pltpu.sync_copy(ones, counts.at[hbuf], add=True)
            return 0

        lax.fori_loop(0, n_sc_ch, scatter_chunk, 0)
        plsc.subcore_barrier()


        ostart = jnp.minimum(wid * own, n_total - own)

        def gather_chunk(j, _):
            base = ostart + jnp.minimum(j * C, own - C)
            for r in range(4):
                pltpu.sync_copy(pts_hbm.at[pl.ds(r * n_total + base, C)],
                                pbuf.at[pl.ds(r * C, C)])

            def hash_i(i, _):
                _hash_chunk(pbuf, hbuf, base, i, n_batch, n_total)
                o = i * L
                hv = hbuf[pl.ds(o, L)]
                for k in range(6):
                    ilist[pl.ds(k * C + o, L)] = (hv + DELTAS[k]) & MASK
                return 0

            lax.fori_loop(0, C // L, hash_i, 0)
            pltpu.sync_copy(counts.at[ilist], taps)

            def patt_i(i, _):
                o = i * L
                pv = jnp.full((L,), 1, jnp.int32)
                for k in range(6):
                    tv = taps[pl.ds(k * C + o, L)]
                    pv = pv | jnp.where(tv > 0.0, 1 << (k + 1), 0)
                patt[pl.ds(o, L)] = pv
                return 0

            lax.fori_loop(0, C // L, patt_i, 0)
            pltpu.sync_copy(patt, idx_hbm.at[pl.ds(base, C)])
            return 0

        lax.fori_loop(0, n_own_ch, gather_chunk, 0)

    return pl.kernel(
        body,
        out_type=jax.ShapeDtypeStruct((m_out,), jnp.int32),
        mesh=mesh,
        scratch_types=[
            pltpu.VMEM((4 * C,), jnp.float32),
            pltpu.VMEM((C,), jnp.int32),
            pltpu.VMEM((C,), jnp.float32),
            pltpu.VMEM((C,), jnp.float32),
            pltpu.VMEM((6 * C,), jnp.int32),
            pltpu.VMEM((6 * C,), jnp.float32),
            pltpu.VMEM((C,), jnp.int32),
            pltpu.VMEM_SHARED((M_BUCKETS,), jnp.float32),
        ],
    )


def _feats_body(idx_ref, w0, b0, w1, b1, w2, b2, w3, b3, out_ref, lut_ref):
    @pl.when(pl.program_id(0) == 0)
    def _():
        p = lax.broadcasted_iota(jnp.int32, (128, 7), 0)
        k = lax.broadcasted_iota(jnp.int32, (128, 7), 1)
        pat = (((p >> k) & 1).astype(jnp.float32)) * 0.5
        x = jnp.maximum(jnp.dot(pat, w0[...], preferred_element_type=jnp.float32)
                        + b0[...], 0.0)
        x = jnp.maximum(jnp.dot(x, w1[...], preferred_element_type=jnp.float32)
                        + b1[...], 0.0)
        x = jnp.maximum(jnp.dot(x, w2[...], preferred_element_type=jnp.float32)
                        + b2[...], 0.0)
        lut_ref[...] = (jnp.dot(x, w3[...], preferred_element_type=jnp.float32)
                        + b3[...])

    idxv = idx_ref[0, 0, :].reshape(-1, 1)
    oh = (idxv == lax.broadcasted_iota(jnp.int32, (idxv.shape[0], 128), 1)
          ).astype(jnp.float32)
    out_ref[...] = jnp.dot(oh, lut_ref[...], preferred_element_type=jnp.float32)


def _cs_body(x_ref, o_ref):
    m = lax.broadcasted_iota(jnp.int32, x_ref.shape, 1) % 4
    q = jnp.where(m == 3, 1.0, 0.1).astype(jnp.float32)
    o_ref[...] = jnp.floor(x_ref[...] / q) * q


def kernel(pcds_4d_batch, W0, b0, W1, b1, W2, b2, W3, b3):
    B, N, _ = pcds_4d_batch.shape
    n_total = B * N
    flat = pcds_4d_batch.reshape(n_total, 4)
    pts_flat = flat.T.reshape(-1)

    BLK = 4096
    nb = (n_total + BLK - 1) // BLK
    m_out = nb * BLK

    patt_idx = _make_sc_kernel(n_total, N, m_out)(pts_flat)

    feats = pl.pallas_call(
        _feats_body,
        grid=(nb,),
        in_specs=[
            pl.BlockSpec((1, 1, BLK), lambda i: (i, 0, 0)),
            pl.BlockSpec((7, 32), lambda i: (0, 0)),
            pl.BlockSpec((1, 32), lambda i: (0, 0)),
            pl.BlockSpec((32, 128), lambda i: (0, 0)),
            pl.BlockSpec((1, 128), lambda i: (0, 0)),
            pl.BlockSpec((128, 32), lambda i: (0, 0)),
            pl.BlockSpec((1, 32), lambda i: (0, 0)),
            pl.BlockSpec((32, 16), lambda i: (0, 0)),
            pl.BlockSpec((1, 16), lambda i: (0, 0)),
        ],
        out_specs=pl.BlockSpec((BLK, 16), lambda i: (i, 0)),
        out_shape=jax.ShapeDtypeStruct((n_total, 16), jnp.float32),
        scratch_shapes=[pltpu.VMEM((128, 16), jnp.float32)],
    )(patt_idx.reshape(nb, 1, BLK), W0, b0.reshape(1, 32), W1, b1.reshape(1, 128),
      W2, b2.reshape(1, 32), W3, b3.reshape(1, 16))
    point_feats = feats

    rows = n_total * 4 // 128
    RB = 4096
    nrb = (rows + RB - 1) // RB
    cs = pl.pallas_call(
        _cs_body,
        grid=(nrb,),
        in_specs=[pl.BlockSpec((RB, 128), lambda i: (i, 0))],
        out_specs=pl.BlockSpec((RB, 128), lambda i: (i, 0)),
        out_shape=jax.ShapeDtypeStruct((rows, 128), jnp.float32),
    )(flat.reshape(rows, 128))
    coords_scaled = cs.reshape(n_total, 4)

    return point_feats, coords_scaled
